# Initial kernel scaffold; baseline (speedup 1.0000x reference)
import jax
import jax.numpy as jnp
from jax.experimental import pallas as pl


def kernel(x, c1_rhs_e, c1_rhs_o, c1_brow, c1_sh_e, c1_sh_o, c2_rhs_e, c2_rhs_o, c2_brow, c2_sh_e, c2_sh_o, fc_w1t, fc_b1row, fc_w2t, fc_b2row, fc_w3t, fc_b3row):
    raise NotImplementedError("write your pallas kernel here")


# speedup vs baseline: 2.1062x; 2.1062x over previous
import jax
import jax.numpy as jnp
from jax.experimental import pallas as pl
from jax.experimental.pallas import tpu as pltpu

_NB = 16
_HIN1, _KH1, _CW1, _CWP1 = 159, 20, 159, 140
_HIN2, _KH2, _CW2, _CWP2 = 70, 35, 140, 72
_HO1 = _HIN1 - _KH1 + 1
_HP1 = _HO1 // 2
_HO2 = _HIN2 - _KH2 + 1
_HP2 = _HO2 // 2


def _conv_body(x_ref, rhs1_ref, b1_ref, rhs2_ref, b2_ref, o_ref,
               xb_ref, acc1_ref, pool1_ref, acc2_ref):
    nb = _NB
    xb_ref[...] = x_ref[...].astype(jnp.bfloat16).reshape(_HIN1 * nb, _CW1)

    m1 = _HO1 * nb
    acc1_ref[...] = jnp.dot(xb_ref[0:m1, :], rhs1_ref[0],
                            preferred_element_type=jnp.float32)
    for kh in range(1, _KH1):
        acc1_ref[...] += jnp.dot(xb_ref[kh * nb:kh * nb + m1, :], rhs1_ref[kh],
                                 preferred_element_type=jnp.float32)

    a = acc1_ref[...].reshape(_HP1, 2 * nb, 2 * _CWP1)
    m = jnp.maximum(a[:, 0:nb, :], a[:, nb:2 * nb, :])
    m = jnp.maximum(m[:, :, 0:_CWP1], m[:, :, _CWP1:2 * _CWP1])
    z = jnp.maximum(m + b1_ref[...], 0.0)
    pool1_ref[...] = z.astype(jnp.bfloat16).reshape(_HP1 * nb, _CW2)

    m2 = _HO2 * nb
    acc2_ref[...] = jnp.dot(pool1_ref[0:m2, :], rhs2_ref[0],
                            preferred_element_type=jnp.float32)
    for kh in range(1, _KH2):
        acc2_ref[...] += jnp.dot(pool1_ref[kh * nb:kh * nb + m2, :],
                                 rhs2_ref[kh],
                                 preferred_element_type=jnp.float32)

    a2 = acc2_ref[...].reshape(_HP2, 2 * nb, 2 * _CWP2)
    m2p = jnp.maximum(a2[:, 0:nb, :], a2[:, nb:2 * nb, :])
    m2p = jnp.maximum(m2p[:, :, 0:_CWP2], m2p[:, :, _CWP2:2 * _CWP2])
    o_ref[...] = jnp.maximum(m2p + b2_ref[...], 0.0)


def _fc_body(h_ref, w1_ref, b1_ref, w2_ref, b2_ref, w3_ref, b3_ref, o_ref):
    h1 = jnp.maximum(
        jnp.dot(h_ref[...], w1_ref[...], preferred_element_type=jnp.float32)
        + b1_ref[...], 0.0)
    h2 = jnp.maximum(
        jnp.dot(h1, w2_ref[...], preferred_element_type=jnp.float32)
        + b2_ref[...], 0.0)
    o_ref[...] = (
        jnp.dot(h2, w3_ref[...], preferred_element_type=jnp.float32)
        + b3_ref[...])


def kernel(x, c1_rhs_e, c1_rhs_o, c1_brow, c1_sh_e, c1_sh_o,
           c2_rhs_e, c2_rhs_o, c2_brow, c2_sh_e, c2_sh_o,
           fc_w1t, fc_b1row, fc_w2t, fc_b2row, fc_w3t, fc_b3row):
    del c1_sh_e, c1_sh_o, c2_sh_e, c2_sh_o
    n = x.shape[0]
    nb = _NB
    xh = x.reshape(n, _HIN1, _CW1).transpose(1, 0, 2)
    rhs1 = jnp.concatenate([c1_rhs_e, c1_rhs_o], axis=2).astype(jnp.bfloat16)
    rhs2 = jnp.concatenate([c2_rhs_e, c2_rhs_o], axis=2).astype(jnp.bfloat16)

    conv_out = pl.pallas_call(
        _conv_body,
        out_shape=jax.ShapeDtypeStruct((_HP2, n, _CWP2), jnp.float32),
        grid=(n // nb,),
        in_specs=[
            pl.BlockSpec((_HIN1, nb, _CW1), lambda i: (0, i, 0)),
            pl.BlockSpec((_KH1, _CW1, 2 * _CWP1), lambda i: (0, 0, 0)),
            pl.BlockSpec((1, _CWP1), lambda i: (0, 0)),
            pl.BlockSpec((_KH2, _CW2, 2 * _CWP2), lambda i: (0, 0, 0)),
            pl.BlockSpec((1, _CWP2), lambda i: (0, 0)),
        ],
        out_specs=pl.BlockSpec((_HP2, nb, _CWP2), lambda i: (0, i, 0)),
        scratch_shapes=[
            pltpu.VMEM((_HIN1 * nb, _CW1), jnp.bfloat16),
            pltpu.VMEM((_HO1 * nb, 2 * _CWP1), jnp.float32),
            pltpu.VMEM((_HP1 * nb, _CW2), jnp.bfloat16),
            pltpu.VMEM((_HO2 * nb, 2 * _CWP2), jnp.float32),
        ],
        compiler_params=pltpu.CompilerParams(
            dimension_semantics=("parallel",),
            vmem_limit_bytes=100 * 1024 * 1024,
        ),
    )(xh, rhs1, c1_brow, rhs2, c2_brow)

    h = conv_out.transpose(1, 0, 2).reshape(n, _HP2 * _CWP2)

    bn = min(128, n)
    return pl.pallas_call(
        _fc_body,
        out_shape=jax.ShapeDtypeStruct((n, fc_w3t.shape[1]), jnp.float32),
        grid=(n // bn,),
        in_specs=[
            pl.BlockSpec((bn, _HP2 * _CWP2), lambda i: (i, 0)),
            pl.BlockSpec(fc_w1t.shape, lambda i: (0, 0)),
            pl.BlockSpec(fc_b1row.shape, lambda i: (0, 0)),
            pl.BlockSpec(fc_w2t.shape, lambda i: (0, 0)),
            pl.BlockSpec(fc_b2row.shape, lambda i: (0, 0)),
            pl.BlockSpec(fc_w3t.shape, lambda i: (0, 0)),
            pl.BlockSpec(fc_b3row.shape, lambda i: (0, 0)),
        ],
        out_specs=pl.BlockSpec((bn, fc_w3t.shape[1]), lambda i: (i, 0)),
        compiler_params=pltpu.CompilerParams(
            dimension_semantics=("parallel",),
        ),
    )(h, fc_w1t, fc_b1row, fc_w2t, fc_b2row, fc_w3t, fc_b3row)


# speedup vs baseline: 2.4113x; 1.1449x over previous
import jax
import jax.numpy as jnp
from jax.experimental import pallas as pl
from jax.experimental.pallas import tpu as pltpu

_NB = 16
_HIN1, _KH1, _CW1, _CWP1 = 159, 20, 159, 140
_HIN2, _KH2, _CW2, _CWP2 = 70, 35, 140, 72
_HO1 = _HIN1 - _KH1 + 1
_HP1 = _HO1 // 2
_HO2 = _HIN2 - _KH2 + 1
_HP2 = _HO2 // 2
_NJ1 = _KH1 + 3
_J1A = 12
_K2A = 18


def _conv_body(x4_ref, rhs1a_ref, rhs1b_ref, b1_ref, rhs2a_ref, rhs2b_ref,
               b2_ref, o_ref,
               xc1a_ref, xc1b_ref, acc1_ref, pool1_ref,
               xc2a_ref, xc2b_ref, acc2_ref):
    nb = _NB
    nq = _HP1 // 2
    m1 = nq * nb
    for j in range(_J1A):
        r, q0 = j % 4, j // 4
        xc1a_ref[:, j * _CW1:(j + 1) * _CW1] = (
            x4_ref[r, q0:q0 + nq, :, :].reshape(m1, _CW1))
    acc1_ref[...] = jnp.dot(xc1a_ref[...], rhs1a_ref[...],
                            preferred_element_type=jnp.float32)
    for j in range(_J1A, _NJ1):
        r, q0 = j % 4, j // 4
        xc1b_ref[:, (j - _J1A) * _CW1:(j - _J1A + 1) * _CW1] = (
            x4_ref[r, q0:q0 + nq, :, :].reshape(m1, _CW1))
    acc1_ref[...] += jnp.dot(xc1b_ref[...], rhs1b_ref[...],
                             preferred_element_type=jnp.float32)

    v = acc1_ref[...]
    cw = 2 * _CWP1
    ma = jnp.maximum(v[:, 0:cw], v[:, cw:2 * cw])
    mb = jnp.maximum(v[:, 2 * cw:3 * cw], v[:, 3 * cw:4 * cw])
    ma = jnp.maximum(ma[:, 0:_CWP1], ma[:, _CWP1:cw])
    mb = jnp.maximum(mb[:, 0:_CWP1], mb[:, _CWP1:cw])
    za = jnp.maximum(ma + b1_ref[...], 0.0).astype(jnp.bfloat16)
    zb = jnp.maximum(mb + b1_ref[...], 0.0).astype(jnp.bfloat16)
    c = jnp.concatenate([za.reshape(nq, 1, nb, _CW2),
                         zb.reshape(nq, 1, nb, _CW2)], axis=1)
    pool1_ref[...] = c.reshape(_HP1 * nb, _CW2)

    m2 = _HO2 * nb
    for kh in range(_K2A):
        xc2a_ref[:, kh * _CW2:(kh + 1) * _CW2] = (
            pool1_ref[kh * nb:kh * nb + m2, :])
    acc2_ref[...] = jnp.dot(xc2a_ref[...], rhs2a_ref[...],
                            preferred_element_type=jnp.float32)
    for kh in range(_K2A, _KH2):
        xc2b_ref[:, (kh - _K2A) * _CW2:(kh - _K2A + 1) * _CW2] = (
            pool1_ref[kh * nb:kh * nb + m2, :])
    acc2_ref[...] += jnp.dot(xc2b_ref[...], rhs2b_ref[...],
                             preferred_element_type=jnp.float32)

    a2 = acc2_ref[...].reshape(_HP2, 2 * nb, 2 * _CWP2)
    m2p = jnp.maximum(a2[:, 0:nb, :], a2[:, nb:2 * nb, :])
    m2p = jnp.maximum(m2p[:, :, 0:_CWP2], m2p[:, :, _CWP2:2 * _CWP2])
    o_ref[...] = jnp.maximum(m2p + b2_ref[...], 0.0)


def _fc_body(h_ref, w1_ref, b1_ref, w2_ref, b2_ref, w3_ref, b3_ref, o_ref):
    h1 = jnp.maximum(
        jnp.dot(h_ref[...], w1_ref[...], preferred_element_type=jnp.float32)
        + b1_ref[...], 0.0)
    h2 = jnp.maximum(
        jnp.dot(h1, w2_ref[...], preferred_element_type=jnp.float32)
        + b2_ref[...], 0.0)
    o_ref[...] = (
        jnp.dot(h2, w3_ref[...], preferred_element_type=jnp.float32)
        + b3_ref[...])


def kernel(x, c1_rhs_e, c1_rhs_o, c1_brow, c1_sh_e, c1_sh_o,
           c2_rhs_e, c2_rhs_o, c2_brow, c2_sh_e, c2_sh_o,
           fc_w1t, fc_b1row, fc_w2t, fc_b2row, fc_w3t, fc_b3row):
    del c1_sh_e, c1_sh_o, c2_sh_e, c2_sh_o
    n = x.shape[0]
    nb = _NB
    xp = jnp.pad(x.reshape(n, _HIN1, _CW1), ((0, 0), (0, 1), (0, 0)))
    x4 = xp.astype(jnp.bfloat16).reshape(n, 40, 4, _CW1).transpose(2, 1, 0, 3)

    rhs1 = jnp.concatenate([c1_rhs_e, c1_rhs_o], axis=2)
    zp = lambda k: jnp.zeros((k, _CW1, 2 * _CWP1), rhs1.dtype)
    rhs1p = jnp.concatenate(
        [jnp.concatenate([zp(p), rhs1, zp(3 - p)], axis=0)
         for p in range(4)], axis=2)
    rhs1p = rhs1p.astype(jnp.bfloat16)
    rhs1p = rhs1p.reshape(_NJ1 * _CW1, 8 * _CWP1)
    rhs1a, rhs1b = rhs1p[:_J1A * _CW1], rhs1p[_J1A * _CW1:]
    rhs2 = jnp.concatenate([c2_rhs_e, c2_rhs_o], axis=2).astype(jnp.bfloat16)
    rhs2 = rhs2.reshape(_KH2 * _CW2, 2 * _CWP2)
    rhs2a, rhs2b = rhs2[:_K2A * _CW2], rhs2[_K2A * _CW2:]

    conv_out = pl.pallas_call(
        _conv_body,
        out_shape=jax.ShapeDtypeStruct((_HP2, n, _CWP2), jnp.float32),
        grid=(n // nb,),
        in_specs=[
            pl.BlockSpec((4, 40, nb, _CW1), lambda i: (0, 0, i, 0)),
            pl.BlockSpec((_J1A * _CW1, 8 * _CWP1), lambda i: (0, 0)),
            pl.BlockSpec(((_NJ1 - _J1A) * _CW1, 8 * _CWP1), lambda i: (0, 0)),
            pl.BlockSpec((1, _CWP1), lambda i: (0, 0)),
            pl.BlockSpec((_K2A * _CW2, 2 * _CWP2), lambda i: (0, 0)),
            pl.BlockSpec(((_KH2 - _K2A) * _CW2, 2 * _CWP2), lambda i: (0, 0)),
            pl.BlockSpec((1, _CWP2), lambda i: (0, 0)),
        ],
        out_specs=pl.BlockSpec((_HP2, nb, _CWP2), lambda i: (0, i, 0)),
        scratch_shapes=[
            pltpu.VMEM((_HP1 // 2 * nb, _J1A * _CW1), jnp.bfloat16),
            pltpu.VMEM((_HP1 // 2 * nb, (_NJ1 - _J1A) * _CW1), jnp.bfloat16),
            pltpu.VMEM((_HP1 // 2 * nb, 8 * _CWP1), jnp.float32),
            pltpu.VMEM((_HP1 * nb, _CW2), jnp.bfloat16),
            pltpu.VMEM((_HO2 * nb, _K2A * _CW2), jnp.bfloat16),
            pltpu.VMEM((_HO2 * nb, (_KH2 - _K2A) * _CW2), jnp.bfloat16),
            pltpu.VMEM((_HO2 * nb, 2 * _CWP2), jnp.float32),
        ],
        compiler_params=pltpu.CompilerParams(
            dimension_semantics=("parallel",),
            vmem_limit_bytes=100 * 1024 * 1024,
        ),
    )(x4, rhs1a, rhs1b, c1_brow, rhs2a, rhs2b, c2_brow)

    h = conv_out.transpose(1, 0, 2).reshape(n, _HP2 * _CWP2)

    bn = min(128, n)
    return pl.pallas_call(
        _fc_body,
        out_shape=jax.ShapeDtypeStruct((n, fc_w3t.shape[1]), jnp.float32),
        grid=(n // bn,),
        in_specs=[
            pl.BlockSpec((bn, _HP2 * _CWP2), lambda i: (i, 0)),
            pl.BlockSpec(fc_w1t.shape, lambda i: (0, 0)),
            pl.BlockSpec(fc_b1row.shape, lambda i: (0, 0)),
            pl.BlockSpec(fc_w2t.shape, lambda i: (0, 0)),
            pl.BlockSpec(fc_b2row.shape, lambda i: (0, 0)),
            pl.BlockSpec(fc_w3t.shape, lambda i: (0, 0)),
            pl.BlockSpec(fc_b3row.shape, lambda i: (0, 0)),
        ],
        out_specs=pl.BlockSpec((bn, fc_w3t.shape[1]), lambda i: (i, 0)),
        compiler_params=pltpu.CompilerParams(
            dimension_semantics=("parallel",),
        ),
    )(h, fc_w1t, fc_b1row, fc_w2t, fc_b2row, fc_w3t, fc_b3row)
